# baseline (speedup 1.0000x reference)
import functools

import jax
import jax.numpy as jnp
from jax import lax
from jax.experimental import pallas as pl
from jax.experimental.pallas import tpu as pltpu
from jax.experimental.pallas import tpu_sc as plsc

V = 1000000
D = 64
BATCH = 16384
HIST = 50
B_TOTAL = BATCH * HIST
NC = 2
NS = 16
PLANES_PER_CORE = D // NC
BCH = 2048
CHUNKS_PER_H = BATCH // BCH
UNITS = HIST * CHUNKS_PER_H
UNITS_PER_TILE = UNITS // NS
IDX_PER_TILE = UNITS_PER_TILE * BCH
SEG = V // 8

_mesh = plsc.VectorSubcoreMesh(core_axis_name="c", subcore_axis_name="s")


@functools.partial(
    pl.kernel,
    mesh=_mesh,
    out_type=jax.ShapeDtypeStruct((HIST, D, BATCH), jnp.float32),
    scratch_types=[
        pltpu.VMEM((IDX_PER_TILE,), jnp.int32),
        pltpu.VMEM((BCH,), jnp.float32),
        pltpu.VMEM((BCH,), jnp.float32),
        pltpu.VMEM_SHARED((V,), jnp.float32),
        pltpu.SemaphoreType.DMA,
        pltpu.SemaphoreType.DMA,
        pltpu.SemaphoreType.DMA,
        pltpu.SemaphoreType.DMA,
    ],
    compiler_params=pltpu.CompilerParams(use_tc_tiling_on_sc=False),
)
def _sc_plane_gather(idx_hbm, table_t, out_hbm, idx_v, buf0, buf1, plane,
                     g0, g1, w0, w1):
    c = lax.axis_index("c")
    s = lax.axis_index("s")

    pltpu.sync_copy(idx_hbm.at[pl.ds(s * IDX_PER_TILE, IDX_PER_TILE)], idx_v)

    def fire(u, buf, sem):
        pltpu.async_copy(plane.at[idx_v.at[pl.ds(u * BCH, BCH)]], buf, sem)

    def drain(u, buf, sem):
        pltpu.make_async_copy(
            plane.at[idx_v.at[pl.ds(u * BCH, BCH)]], buf, sem).wait()

    def wb_copy(d, u, buf, sem):
        gid = s * UNITS_PER_TILE + u
        h = gid // CHUNKS_PER_H
        j = gid % CHUNKS_PER_H
        return pltpu.make_async_copy(
            buf, out_hbm.at[h, d, pl.ds(j * BCH, BCH)], sem)

    def plane_body(i, carry):
        d = PLANES_PER_CORE * c + i

        @pl.when(s < 8)
        def _load():
            pltpu.sync_copy(table_t.at[d, pl.ds(s * SEG, SEG)],
                            plane.at[pl.ds(s * SEG, SEG)])

        plsc.subcore_barrier()

        fire(0, buf0, g0)

        def pair(q, cc):
            u = 2 * q

            @pl.when(q > 0)
            def _wait_wb1():
                wb_copy(d, u - 1, buf1, w1).wait()

            fire(u + 1, buf1, g1)
            drain(u, buf0, g0)
            wb_copy(d, u, buf0, w0).start()
            wb_copy(d, u, buf0, w0).wait()
            fire(u + 2, buf0, g0)
            drain(u + 1, buf1, g1)
            wb_copy(d, u + 1, buf1, w1).start()
            return cc

        lax.fori_loop(0, (UNITS_PER_TILE - 1) // 2, pair, 0)

        drain(UNITS_PER_TILE - 1, buf0, g0)
        wb_copy(d, UNITS_PER_TILE - 2, buf1, w1).wait()
        wb_copy(d, UNITS_PER_TILE - 1, buf0, w0).start()
        wb_copy(d, UNITS_PER_TILE - 1, buf0, w0).wait()

        plsc.subcore_barrier()
        return carry

    lax.fori_loop(0, PLANES_PER_CORE, plane_body, 0)


def kernel(indices, table):
    idx_t = jnp.transpose(indices).reshape(B_TOTAL).astype(jnp.int32)
    table_t = jnp.transpose(table)
    out_t = _sc_plane_gather(idx_t, table_t)
    return jnp.transpose(out_t, (2, 0, 1))


# speedup vs baseline: 1.0120x; 1.0120x over previous
import functools

import jax
import jax.numpy as jnp
from jax import lax
from jax.experimental import pallas as pl
from jax.experimental.pallas import tpu as pltpu
from jax.experimental.pallas import tpu_sc as plsc

V = 1000000
D = 64
BATCH = 16384
HIST = 50
B_TOTAL = BATCH * HIST
NC = 2
NS = 16
PLANES_PER_CORE = D // NC
BCH = 2048
CHUNKS_PER_H = BATCH // BCH
UNITS_PER_TILE = (HIST * CHUNKS_PER_H) // NS
IDX_PER_TILE = UNITS_PER_TILE * BCH
SEG = V // 8
NBUF = 6
AHEAD = 2

_mesh = plsc.VectorSubcoreMesh(core_axis_name="c", subcore_axis_name="s")


@functools.partial(
    pl.kernel,
    mesh=_mesh,
    out_type=jax.ShapeDtypeStruct((HIST, D, BATCH), jnp.float32),
    scratch_types=[
        pltpu.VMEM((IDX_PER_TILE,), jnp.int32),
        pltpu.VMEM((NBUF, BCH), jnp.float32),
        pltpu.VMEM_SHARED((V,), jnp.float32),
        pltpu.SemaphoreType.DMA((NBUF,)),
        pltpu.SemaphoreType.DMA((NBUF,)),
        pltpu.SemaphoreType.DMA,
    ],
    compiler_params=pltpu.CompilerParams(use_tc_tiling_on_sc=False),
)
def _sc_plane_gather(idx_hbm, table_lin, out_hbm, idx_v, bufs, plane0,
                     gsem, wsem, lsem):
    c = lax.axis_index("c")
    s = lax.axis_index("s")

    pltpu.sync_copy(idx_hbm.at[pl.ds(s * IDX_PER_TILE, IDX_PER_TILE)], idx_v)

    def load_plane(d, pbuf):
        pltpu.async_copy(
            table_lin.at[pl.ds(d * V + s * SEG, SEG)],
            pbuf.at[pl.ds(s * SEG, SEG)],
            lsem,
        )

    def wait_load(d, pbuf):
        pltpu.make_async_copy(
            table_lin.at[pl.ds(d * V + s * SEG, SEG)],
            pbuf.at[pl.ds(s * SEG, SEG)],
            lsem,
        ).wait()

    def gather_copy(pbuf, u, b):
        return pltpu.make_async_copy(
            pbuf.at[idx_v.at[pl.ds(u * BCH, BCH)]],
            bufs.at[b],
            gsem.at[b],
        )

    def wb_copy(d, u, b):
        gid = s * UNITS_PER_TILE + u
        h = gid // CHUNKS_PER_H
        j = gid % CHUNKS_PER_H
        return pltpu.make_async_copy(
            bufs.at[b], out_hbm.at[h, d, pl.ds(j * BCH, BCH)], wsem.at[b])

    def run_plane(d, pbuf):
        for u in range(AHEAD):
            gather_copy(pbuf, u, u % NBUF).start()
        for u in range(UNITS_PER_TILE):
            b = u % NBUF
            un = u + AHEAD
            if un < UNITS_PER_TILE:
                bn = un % NBUF
                if un >= NBUF:
                    wb_copy(d, un - NBUF, bn).wait()
                gather_copy(pbuf, un, bn).start()
            gather_copy(pbuf, u, b).wait()
            wb_copy(d, u, b).start()
        for u in range(UNITS_PER_TILE - NBUF, UNITS_PER_TILE):
            wb_copy(d, u, u % NBUF).wait()

    d0 = PLANES_PER_CORE * c

    def body(p, carry):
        d = d0 + p

        @pl.when(s < 8)
        def _load():
            load_plane(d, plane0)
            wait_load(d, plane0)

        plsc.subcore_barrier()
        run_plane(d, plane0)
        plsc.subcore_barrier()
        return carry

    lax.fori_loop(0, PLANES_PER_CORE, body, 0)


def kernel(indices, table):
    idx_t = jnp.transpose(indices).reshape(B_TOTAL).astype(jnp.int32)
    table_lin = jnp.transpose(table).reshape(D * V)
    out_t = _sc_plane_gather(idx_t, table_lin)
    return jnp.transpose(out_t, (2, 0, 1))


# speedup vs baseline: 4.7338x; 4.6776x over previous
import functools

import jax
import jax.numpy as jnp
from jax import lax
from jax.experimental import pallas as pl
from jax.experimental.pallas import tpu as pltpu
from jax.experimental.pallas import tpu_sc as plsc

BATCH = 16384
HIST = 50
D = 64
B_TOTAL = BATCH * HIST
NC = 2
NS = 16
NW = NC * NS
B_PER_W = B_TOTAL // NW
CHUNK = 128
N_CHUNKS = B_PER_W // CHUNK
K = 4
NG = N_CHUNKS // K
GROUP_ROWS = K * CHUNK

_mesh = plsc.VectorSubcoreMesh(core_axis_name="c", subcore_axis_name="s")


@functools.partial(
    pl.kernel,
    mesh=_mesh,
    out_type=jax.ShapeDtypeStruct((B_TOTAL, D), jnp.float32),
    scratch_types=[
        pltpu.VMEM((N_CHUNKS, CHUNK), jnp.int32),
        pltpu.VMEM((GROUP_ROWS, D), jnp.float32),
        pltpu.VMEM((GROUP_ROWS, D), jnp.float32),
        pltpu.SemaphoreType.DMA,
        pltpu.SemaphoreType.DMA,
        pltpu.SemaphoreType.DMA,
        pltpu.SemaphoreType.DMA,
    ],
    compiler_params=pltpu.CompilerParams(use_tc_tiling_on_sc=False),
)
def _sc_gather(idx_hbm, table_hbm, out_hbm, idx_v, rows0, rows1,
               g0, g1, w0, w1):
    wid = lax.axis_index("s") * NC + lax.axis_index("c")
    base = wid * B_PER_W
    rows = [rows0, rows1]
    gsem = [g0, g1]
    wsem = [w0, w1]

    pltpu.sync_copy(idx_hbm.at[pl.ds(wid * N_CHUNKS, N_CHUNKS)], idx_v)

    def fire(group, buf, sem):
        for b in range(K):
            pltpu.async_copy(
                table_hbm.at[idx_v.at[group * K + b]],
                buf.at[pl.ds(b * CHUNK, CHUNK)],
                sem,
            )

    def drain(group, buf, sem):
        for b in range(K):
            pltpu.make_async_copy(
                table_hbm.at[idx_v.at[group * K + b]],
                buf.at[pl.ds(b * CHUNK, CHUNK)],
                sem,
            ).wait()

    def writeback_copy(group, buf, sem):
        return pltpu.make_async_copy(
            buf, out_hbm.at[pl.ds(base + group * GROUP_ROWS, GROUP_ROWS)], sem)

    def start_writeback(group, buf, sem):
        pltpu.async_copy(
            buf, out_hbm.at[pl.ds(base + group * GROUP_ROWS, GROUP_ROWS)], sem)

    NP = NG // 2

    fire(0, rows[0], gsem[0])

    def body(p, carry):
        g = 2 * p

        @pl.when(p > 0)
        def _wait_wb1():
            writeback_copy(g - 1, rows[1], wsem[1]).wait()

        fire(g + 1, rows[1], gsem[1])
        drain(g, rows[0], gsem[0])
        start_writeback(g, rows[0], wsem[0])

        @pl.when(p + 1 < NP)
        def _fire_next_pair():
            writeback_copy(g, rows[0], wsem[0]).wait()
            fire(g + 2, rows[0], gsem[0])

        drain(g + 1, rows[1], gsem[1])
        start_writeback(g + 1, rows[1], wsem[1])
        return carry

    lax.fori_loop(0, NP, body, 0)

    writeback_copy(NG - 2, rows[0], wsem[0]).wait()
    writeback_copy(NG - 1, rows[1], wsem[1]).wait()


def kernel(indices, table):
    idx = indices.reshape(B_TOTAL // CHUNK, CHUNK).astype(jnp.int32)
    out = _sc_gather(idx, table)
    return out.reshape(BATCH, HIST, D)
